# baseline (speedup 1.0000x reference)
import jax
import jax.numpy as jnp
from jax.experimental import pallas as pl
from jax.experimental.pallas import tpu as pltpu

BF = jnp.bfloat16
F32 = jnp.float32

_H, _W = 32, 32
_HO, _WO = 16, 16
_CIN, _COUT, _F = 4, 4, 8
_C0, _C1 = 128, 256
_NB = 2

_M_OFF = {
    ("d0", 0): (0, _C0), ("d0", 1): (128, _C0),
    ("d1", 0): (256, _C1), ("d1", 1): (512, _C1),
    ("a1", 0): (768, _C1), ("a1", 1): (1024, _C1),
    ("a0", 0): (1280, _C0), ("a0", 1): (1408, _C0),
}


def _zero_border(pad_ref):
    _, Hp, Wp, C = pad_ref.shape
    z = jnp.zeros((1, Wp, C), BF)
    pad_ref[0, 0:1, :, :] = z
    pad_ref[0, Hp - 1:Hp, :, :] = z
    zc = jnp.zeros((Hp, 1, C), BF)
    pad_ref[0, :, 0:1, :] = zc
    pad_ref[0, :, Wp - 1:Wp, :] = zc


def _conv3x3(pad_ref, slab_ref, x_hwc, w_flat, bias, H, W, Cin):
    pad_ref[0, 1:H + 1, 1:W + 1, :] = x_hwc
    for k in range(9):
        kh, kw = k // 3, k % 3
        slab_ref[:, k * Cin:(k + 1) * Cin] = (
            pad_ref[0, kh:kh + H, kw:kw + W, :].reshape(H * W, Cin))
    return jnp.dot(slab_ref[...], w_flat,
                   preferred_element_type=F32) + bias


def _mod_block(pad_ref, slab_ref, x_flat, m, w1, b1, w2, b2, H, W, C):
    t = (x_flat + m).astype(BF).reshape(H, W, C)
    h = jnp.maximum(_conv3x3(pad_ref, slab_ref, t, w1, b1, H, W, C), 0.0)
    r = _conv3x3(pad_ref, slab_ref, h.astype(BF).reshape(H, W, C),
                 w2, b2, H, W, C)
    return x_flat + r


def _unet_body(x_ref, y_ref, lwa_ref, lba_ref,
               h0w_ref, h0b_ref, h1w_ref, h1b_ref,
               t1w_ref, t1b_ref, t0w_ref, t0b_ref,
               dec_ref, up_ref,
               d0w1_ref, d0b1_ref, d0w2_ref, d0b2_ref,
               d1w1_ref, d1b1_ref, d1w2_ref, d1b2_ref,
               a1w1_ref, a1b1_ref, a1w2_ref, a1b2_ref,
               a0w1_ref, a0b1_ref, a0w2_ref, a0b2_ref,
               o_ref,
               pad_h0, slab_h0, pad0, slab0, pad1, slab1,
               pad_t, slab_t, cur0, skip0, cur1):
    for p in (pad_h0, pad0, pad1, pad_t):
        _zero_border(p)

    m_all = jnp.dot(y_ref[0], lwa_ref[...],
                    preferred_element_type=F32) + lba_ref[...]

    def mod(tag, j):
        off, c = _M_OFF[(tag, j)]
        return m_all[:, off:off + c]

    cur0[...] = _conv3x3(pad_h0, slab_h0, x_ref[0],
                         h0w_ref[...], h0b_ref[...], _H, _W, _CIN)
    for b in range(_NB):
        cur0[...] = _mod_block(pad0, slab0, cur0[...], mod("d0", b),
                               d0w1_ref[b], d0b1_ref[b],
                               d0w2_ref[b], d0b2_ref[b], _H, _W, _C0)
    skip0[...] = cur0[...]

    res = _conv3x3(pad0, slab0, cur0[...].astype(BF).reshape(_H, _W, _C0),
                   h1w_ref[...], h1b_ref[...], _H, _W, _C0)
    cur1[...] = jnp.dot(dec_ref[...], res.astype(BF),
                        preferred_element_type=F32)
    for b in range(_NB):
        cur1[...] = _mod_block(pad1, slab1, cur1[...], mod("d1", b),
                               d1w1_ref[b], d1b1_ref[b],
                               d1w2_ref[b], d1b2_ref[b], _HO, _WO, _C1)

    for b in range(_NB):
        cur1[...] = _mod_block(pad1, slab1, cur1[...], mod("a1", b),
                               a1w1_ref[b], a1b1_ref[b],
                               a1w2_ref[b], a1b2_ref[b], _HO, _WO, _C1)
    tin = jnp.dot(up_ref[...], cur1[...].astype(BF),
                  preferred_element_type=F32)
    res = _conv3x3(pad_t, slab_t, tin.astype(BF).reshape(_H, _W, _C1),
                   t1w_ref[...], t1b_ref[...], _H, _W, _C1)
    cur0[...] = res + skip0[...]

    for b in range(_NB):
        cur0[...] = _mod_block(pad0, slab0, cur0[...], mod("a0", b),
                               a0w1_ref[b], a0b1_ref[b],
                               a0w2_ref[b], a0b2_ref[b], _H, _W, _C0)
    res = _conv3x3(pad0, slab0, cur0[...].astype(BF).reshape(_H, _W, _C0),
                   t0w_ref[...], t0b_ref[...], _H, _W, _C0)
    o_ref[0] = res.reshape(_H, _W, _COUT)


def _decimation_matrix(H, W, Ho, Wo):
    rows = jnp.arange(Ho * Wo, dtype=jnp.int32)
    src = 2 * (rows // Wo) * W + 2 * (rows % Wo)
    return (jnp.arange(H * W, dtype=jnp.int32)[None, :] == src[:, None]
            ).astype(BF)


def _upsample_matrix(Hi, Wi, Ht, Wt):
    rows = jnp.arange(Ht * Wt, dtype=jnp.int32)
    src = ((rows // Wt) // 2) * Wi + (rows % Wt) // 2
    return (jnp.arange(Hi * Wi, dtype=jnp.int32)[None, :] == src[:, None]
            ).astype(BF)


def kernel(x, y, head_w_0, head_b_0, head_w_1, head_b_1, tail_w_0, tail_b_0, tail_w_1, tail_b_1, desc_0_0_lw, desc_0_0_lb, desc_0_0_w1, desc_0_0_b1, desc_0_0_w2, desc_0_0_b2, desc_0_1_lw, desc_0_1_lb, desc_0_1_w1, desc_0_1_b1, desc_0_1_w2, desc_0_1_b2, desc_1_0_lw, desc_1_0_lb, desc_1_0_w1, desc_1_0_b1, desc_1_0_w2, desc_1_0_b2, desc_1_1_lw, desc_1_1_lb, desc_1_1_w1, desc_1_1_b1, desc_1_1_w2, desc_1_1_b2, asc_0_0_lw, asc_0_0_lb, asc_0_0_w1, asc_0_0_b1, asc_0_0_w2, asc_0_0_b2, asc_0_1_lw, asc_0_1_lb, asc_0_1_w1, asc_0_1_b1, asc_0_1_w2, asc_0_1_b2, asc_1_0_lw, asc_1_0_lb, asc_1_0_w1, asc_1_0_b1, asc_1_0_w2, asc_1_0_b2, asc_1_1_lw, asc_1_1_lb, asc_1_1_w1, asc_1_1_b1, asc_1_1_w2, asc_1_1_b2):
    N = x.shape[0]

    x_nhwc = jnp.transpose(x, (0, 2, 3, 1)).astype(BF)
    y3 = y.astype(F32).reshape(N, 1, _F)

    lws = [desc_0_0_lw, desc_0_1_lw, desc_1_0_lw, desc_1_1_lw,
           asc_1_0_lw, asc_1_1_lw, asc_0_0_lw, asc_0_1_lw]
    lbs = [desc_0_0_lb, desc_0_1_lb, desc_1_0_lb, desc_1_1_lb,
           asc_1_0_lb, asc_1_1_lb, asc_0_0_lb, asc_0_1_lb]
    lwa = jnp.concatenate([w.astype(F32) for w in lws], axis=1)
    lba = jnp.concatenate([b.astype(F32) for b in lbs]).reshape(1, -1)

    cw = lambda w, k, c: w.reshape(9 * k, c).astype(BF)
    cb = lambda b: b.astype(F32).reshape(1, -1)
    stk_w = lambda ws, k, c: jnp.stack([cw(w, k, c) for w in ws])
    stk_b = lambda bs: jnp.stack([cb(b) for b in bs])

    h0w, h0b = cw(head_w_0, _CIN, _C0), cb(head_b_0)
    h1w, h1b = cw(head_w_1, _C0, _C1), cb(head_b_1)
    t1w, t1b = cw(tail_w_1, _C1, _C0), cb(tail_b_1)
    t0w, t0b = cw(tail_w_0, _C0, _COUT), cb(tail_b_0)

    d0w1 = stk_w([desc_0_0_w1, desc_0_1_w1], _C0, _C0)
    d0w2 = stk_w([desc_0_0_w2, desc_0_1_w2], _C0, _C0)
    d0b1 = stk_b([desc_0_0_b1, desc_0_1_b1])
    d0b2 = stk_b([desc_0_0_b2, desc_0_1_b2])
    d1w1 = stk_w([desc_1_0_w1, desc_1_1_w1], _C1, _C1)
    d1w2 = stk_w([desc_1_0_w2, desc_1_1_w2], _C1, _C1)
    d1b1 = stk_b([desc_1_0_b1, desc_1_1_b1])
    d1b2 = stk_b([desc_1_0_b2, desc_1_1_b2])
    a1w1 = stk_w([asc_1_0_w1, asc_1_1_w1], _C1, _C1)
    a1w2 = stk_w([asc_1_0_w2, asc_1_1_w2], _C1, _C1)
    a1b1 = stk_b([asc_1_0_b1, asc_1_1_b1])
    a1b2 = stk_b([asc_1_0_b2, asc_1_1_b2])
    a0w1 = stk_w([asc_0_0_w1, asc_0_1_w1], _C0, _C0)
    a0w2 = stk_w([asc_0_0_w2, asc_0_1_w2], _C0, _C0)
    a0b1 = stk_b([asc_0_0_b1, asc_0_1_b1])
    a0b2 = stk_b([asc_0_0_b2, asc_0_1_b2])

    dec = _decimation_matrix(_H, _W, _HO, _WO)
    up = _upsample_matrix(_HO, _WO, _H, _W)

    wspec2 = lambda shape: pl.BlockSpec(shape, lambda n: (0, 0))
    wspec3 = lambda shape: pl.BlockSpec(shape, lambda n: (0, 0, 0))

    in_specs = [
        pl.BlockSpec((1, _H, _W, _CIN), lambda n: (n, 0, 0, 0)),
        pl.BlockSpec((1, 1, _F), lambda n: (n, 0, 0)),
        wspec2(lwa.shape), wspec2(lba.shape),
        wspec2(h0w.shape), wspec2(h0b.shape),
        wspec2(h1w.shape), wspec2(h1b.shape),
        wspec2(t1w.shape), wspec2(t1b.shape),
        wspec2(t0w.shape), wspec2(t0b.shape),
        wspec2(dec.shape), wspec2(up.shape),
        wspec3(d0w1.shape), wspec3(d0b1.shape),
        wspec3(d0w2.shape), wspec3(d0b2.shape),
        wspec3(d1w1.shape), wspec3(d1b1.shape),
        wspec3(d1w2.shape), wspec3(d1b2.shape),
        wspec3(a1w1.shape), wspec3(a1b1.shape),
        wspec3(a1w2.shape), wspec3(a1b2.shape),
        wspec3(a0w1.shape), wspec3(a0b1.shape),
        wspec3(a0w2.shape), wspec3(a0b2.shape),
    ]
    inputs = [x_nhwc, y3, lwa, lba, h0w, h0b, h1w, h1b, t1w, t1b, t0w, t0b,
              dec, up,
              d0w1, d0b1, d0w2, d0b2, d1w1, d1b1, d1w2, d1b2,
              a1w1, a1b1, a1w2, a1b2, a0w1, a0b1, a0w2, a0b2]

    out = pl.pallas_call(
        _unet_body,
        out_shape=jax.ShapeDtypeStruct((N, _H, _W, _COUT), F32),
        grid=(N,),
        in_specs=in_specs,
        out_specs=pl.BlockSpec((1, _H, _W, _COUT), lambda n: (n, 0, 0, 0)),
        scratch_shapes=[
            pltpu.VMEM((1, _H + 2, _W + 2, _CIN), BF),
            pltpu.VMEM((_H * _W, 9 * _CIN), BF),
            pltpu.VMEM((1, _H + 2, _W + 2, _C0), BF),
            pltpu.VMEM((_H * _W, 9 * _C0), BF),
            pltpu.VMEM((1, _HO + 2, _WO + 2, _C1), BF),
            pltpu.VMEM((_HO * _WO, 9 * _C1), BF),
            pltpu.VMEM((1, _H + 2, _W + 2, _C1), BF),
            pltpu.VMEM((_H * _W, 9 * _C1), BF),
            pltpu.VMEM((_H * _W, _C0), F32),
            pltpu.VMEM((_H * _W, _C0), F32),
            pltpu.VMEM((_HO * _WO, _C1), F32),
        ],
        compiler_params=pltpu.CompilerParams(
            dimension_semantics=("parallel",),
            vmem_limit_bytes=100 * 1024 * 1024),
    )(*inputs)

    return jnp.transpose(out, (0, 3, 1, 2))


# speedup vs baseline: 1.4814x; 1.4814x over previous
import jax
import jax.numpy as jnp
from jax.experimental import pallas as pl
from jax.experimental.pallas import tpu as pltpu

BF = jnp.float32
F32 = jnp.float32

_H, _W = 32, 32
_HO, _WO = 16, 16
_CIN, _COUT, _F = 4, 4, 8
_C0, _C1 = 128, 256
_NB = 2

_M_OFF = {
    ("d0", 0): (0, _C0), ("d0", 1): (128, _C0),
    ("d1", 0): (256, _C1), ("d1", 1): (512, _C1),
    ("a1", 0): (768, _C1), ("a1", 1): (1024, _C1),
    ("a0", 0): (1280, _C0), ("a0", 1): (1408, _C0),
}


def _zero_hbands(padh_ref, W):
    M, C = padh_ref.shape
    z = jnp.zeros((W, C), BF)
    padh_ref[0:W, :] = z
    padh_ref[M - W:M, :] = z


def _conv3x3(padh_ref, s_ref, x_flat, w3, bias, H, W, Cin):
    HW = H * W
    padh_ref[W:W + HW, :] = x_flat
    for kh in range(3):
        s_ref[:, kh * Cin:(kh + 1) * Cin] = padh_ref[kh * W:kh * W + HW, :]
    s = s_ref[...]
    t0 = jnp.dot(s, w3[0], preferred_element_type=F32)
    t1 = jnp.dot(s, w3[1], preferred_element_type=F32)
    t2 = jnp.dot(s, w3[2], preferred_element_type=F32)
    Cout = t1.shape[-1]
    zcol = jnp.zeros((H, 1, Cout), F32)
    t0s = jnp.concatenate([zcol, t0.reshape(H, W, Cout)[:, :W - 1, :]],
                          axis=1).reshape(HW, Cout)
    t2s = jnp.concatenate([t2.reshape(H, W, Cout)[:, 1:, :], zcol],
                          axis=1).reshape(HW, Cout)
    return t1 + t0s + t2s + bias


def _mod_block(padh_ref, s_ref, x_flat, m, w31, b1, w32, b2, H, W, C):
    t = (x_flat + m).astype(BF)
    h = jnp.maximum(_conv3x3(padh_ref, s_ref, t, w31, b1, H, W, C), 0.0)
    r = _conv3x3(padh_ref, s_ref, h.astype(BF), w32, b2, H, W, C)
    return x_flat + r


def _unet_body(x_ref, y_ref, lwa_ref, lba_ref,
               h0w_ref, h0b_ref, h1w_ref, h1b_ref,
               t1w_ref, t1b_ref, t0w_ref, t0b_ref,
               dec_ref, up_ref,
               d0w1_ref, d0b1_ref, d0w2_ref, d0b2_ref,
               d1w1_ref, d1b1_ref, d1w2_ref, d1b2_ref,
               a1w1_ref, a1b1_ref, a1w2_ref, a1b2_ref,
               a0w1_ref, a0b1_ref, a0w2_ref, a0b2_ref,
               o_ref,
               ph0, s0h, p0, s0, p1, s1, pt, st, cur0, skip0, cur1):
    _zero_hbands(ph0, _W)
    _zero_hbands(p0, _W)
    _zero_hbands(p1, _WO)
    _zero_hbands(pt, _W)

    m_all = jnp.dot(y_ref[0], lwa_ref[...],
                    preferred_element_type=F32) + lba_ref[...]

    def mod(tag, j):
        off, c = _M_OFF[(tag, j)]
        return m_all[:, off:off + c]

    cur0[...] = _conv3x3(ph0, s0h, x_ref[0].reshape(_H * _W, _CIN),
                         h0w_ref[...], h0b_ref[...], _H, _W, _CIN)
    for b in range(_NB):
        cur0[...] = _mod_block(p0, s0, cur0[...], mod("d0", b),
                               d0w1_ref[b], d0b1_ref[b],
                               d0w2_ref[b], d0b2_ref[b], _H, _W, _C0)
    skip0[...] = cur0[...]

    res = _conv3x3(p0, s0, cur0[...].astype(BF),
                   h1w_ref[...], h1b_ref[...], _H, _W, _C0)
    cur1[...] = jnp.dot(dec_ref[...], res.astype(BF),
                        preferred_element_type=F32)
    for b in range(_NB):
        cur1[...] = _mod_block(p1, s1, cur1[...], mod("d1", b),
                               d1w1_ref[b], d1b1_ref[b],
                               d1w2_ref[b], d1b2_ref[b], _HO, _WO, _C1)

    for b in range(_NB):
        cur1[...] = _mod_block(p1, s1, cur1[...], mod("a1", b),
                               a1w1_ref[b], a1b1_ref[b],
                               a1w2_ref[b], a1b2_ref[b], _HO, _WO, _C1)
    tin = jnp.dot(up_ref[...], cur1[...].astype(BF),
                  preferred_element_type=F32)
    res = _conv3x3(pt, st, tin.astype(BF),
                   t1w_ref[...], t1b_ref[...], _H, _W, _C1)
    cur0[...] = res + skip0[...]

    for b in range(_NB):
        cur0[...] = _mod_block(p0, s0, cur0[...], mod("a0", b),
                               a0w1_ref[b], a0b1_ref[b],
                               a0w2_ref[b], a0b2_ref[b], _H, _W, _C0)
    res = _conv3x3(p0, s0, cur0[...].astype(BF),
                   t0w_ref[...], t0b_ref[...], _H, _W, _C0)
    o_ref[0] = res.reshape(_H, _W, _COUT)


def _decimation_matrix(H, W, Ho, Wo):
    rows = jnp.arange(Ho * Wo, dtype=jnp.int32)
    src = 2 * (rows // Wo) * W + 2 * (rows % Wo)
    return (jnp.arange(H * W, dtype=jnp.int32)[None, :] == src[:, None]
            ).astype(BF)


def _upsample_matrix(Hi, Wi, Ht, Wt):
    rows = jnp.arange(Ht * Wt, dtype=jnp.int32)
    src = ((rows // Wt) // 2) * Wi + (rows % Wt) // 2
    return (jnp.arange(Hi * Wi, dtype=jnp.int32)[None, :] == src[:, None]
            ).astype(BF)


def _w3(w, cin, cout):
    return jnp.transpose(w, (1, 0, 2, 3)).reshape(3, 3 * cin, cout).astype(BF)


def kernel(x, y, head_w_0, head_b_0, head_w_1, head_b_1, tail_w_0, tail_b_0, tail_w_1, tail_b_1, desc_0_0_lw, desc_0_0_lb, desc_0_0_w1, desc_0_0_b1, desc_0_0_w2, desc_0_0_b2, desc_0_1_lw, desc_0_1_lb, desc_0_1_w1, desc_0_1_b1, desc_0_1_w2, desc_0_1_b2, desc_1_0_lw, desc_1_0_lb, desc_1_0_w1, desc_1_0_b1, desc_1_0_w2, desc_1_0_b2, desc_1_1_lw, desc_1_1_lb, desc_1_1_w1, desc_1_1_b1, desc_1_1_w2, desc_1_1_b2, asc_0_0_lw, asc_0_0_lb, asc_0_0_w1, asc_0_0_b1, asc_0_0_w2, asc_0_0_b2, asc_0_1_lw, asc_0_1_lb, asc_0_1_w1, asc_0_1_b1, asc_0_1_w2, asc_0_1_b2, asc_1_0_lw, asc_1_0_lb, asc_1_0_w1, asc_1_0_b1, asc_1_0_w2, asc_1_0_b2, asc_1_1_lw, asc_1_1_lb, asc_1_1_w1, asc_1_1_b1, asc_1_1_w2, asc_1_1_b2):
    N = x.shape[0]

    x_nhwc = jnp.transpose(x, (0, 2, 3, 1)).astype(BF)
    y3 = y.astype(F32).reshape(N, 1, _F)

    lws = [desc_0_0_lw, desc_0_1_lw, desc_1_0_lw, desc_1_1_lw,
           asc_1_0_lw, asc_1_1_lw, asc_0_0_lw, asc_0_1_lw]
    lbs = [desc_0_0_lb, desc_0_1_lb, desc_1_0_lb, desc_1_1_lb,
           asc_1_0_lb, asc_1_1_lb, asc_0_0_lb, asc_0_1_lb]
    lwa = jnp.concatenate([w.astype(F32) for w in lws], axis=1)
    lba = jnp.concatenate([b.astype(F32) for b in lbs]).reshape(1, -1)

    cb = lambda b: b.astype(F32).reshape(1, -1)
    stk_w = lambda ws, k, c: jnp.stack([_w3(w, k, c) for w in ws])
    stk_b = lambda bs: jnp.stack([cb(b) for b in bs])

    h0w, h0b = _w3(head_w_0, _CIN, _C0), cb(head_b_0)
    h1w, h1b = _w3(head_w_1, _C0, _C1), cb(head_b_1)
    t1w, t1b = _w3(tail_w_1, _C1, _C0), cb(tail_b_1)
    t0w, t0b = _w3(tail_w_0, _C0, _COUT), cb(tail_b_0)

    d0w1 = stk_w([desc_0_0_w1, desc_0_1_w1], _C0, _C0)
    d0w2 = stk_w([desc_0_0_w2, desc_0_1_w2], _C0, _C0)
    d0b1 = stk_b([desc_0_0_b1, desc_0_1_b1])
    d0b2 = stk_b([desc_0_0_b2, desc_0_1_b2])
    d1w1 = stk_w([desc_1_0_w1, desc_1_1_w1], _C1, _C1)
    d1w2 = stk_w([desc_1_0_w2, desc_1_1_w2], _C1, _C1)
    d1b1 = stk_b([desc_1_0_b1, desc_1_1_b1])
    d1b2 = stk_b([desc_1_0_b2, desc_1_1_b2])
    a1w1 = stk_w([asc_1_0_w1, asc_1_1_w1], _C1, _C1)
    a1w2 = stk_w([asc_1_0_w2, asc_1_1_w2], _C1, _C1)
    a1b1 = stk_b([asc_1_0_b1, asc_1_1_b1])
    a1b2 = stk_b([asc_1_0_b2, asc_1_1_b2])
    a0w1 = stk_w([asc_0_0_w1, asc_0_1_w1], _C0, _C0)
    a0w2 = stk_w([asc_0_0_w2, asc_0_1_w2], _C0, _C0)
    a0b1 = stk_b([asc_0_0_b1, asc_0_1_b1])
    a0b2 = stk_b([asc_0_0_b2, asc_0_1_b2])

    dec = _decimation_matrix(_H, _W, _HO, _WO)
    up = _upsample_matrix(_HO, _WO, _H, _W)

    wspec2 = lambda shape: pl.BlockSpec(shape, lambda n: (0, 0))
    wspec3 = lambda shape: pl.BlockSpec(shape, lambda n: (0, 0, 0))
    wspec4 = lambda shape: pl.BlockSpec(shape, lambda n: (0, 0, 0, 0))

    in_specs = [
        pl.BlockSpec((1, _H, _W, _CIN), lambda n: (n, 0, 0, 0)),
        pl.BlockSpec((1, 1, _F), lambda n: (n, 0, 0)),
        wspec2(lwa.shape), wspec2(lba.shape),
        wspec3(h0w.shape), wspec2(h0b.shape),
        wspec3(h1w.shape), wspec2(h1b.shape),
        wspec3(t1w.shape), wspec2(t1b.shape),
        wspec3(t0w.shape), wspec2(t0b.shape),
        wspec2(dec.shape), wspec2(up.shape),
        wspec4(d0w1.shape), wspec3(d0b1.shape),
        wspec4(d0w2.shape), wspec3(d0b2.shape),
        wspec4(d1w1.shape), wspec3(d1b1.shape),
        wspec4(d1w2.shape), wspec3(d1b2.shape),
        wspec4(a1w1.shape), wspec3(a1b1.shape),
        wspec4(a1w2.shape), wspec3(a1b2.shape),
        wspec4(a0w1.shape), wspec3(a0b1.shape),
        wspec4(a0w2.shape), wspec3(a0b2.shape),
    ]
    inputs = [x_nhwc, y3, lwa, lba, h0w, h0b, h1w, h1b, t1w, t1b, t0w, t0b,
              dec, up,
              d0w1, d0b1, d0w2, d0b2, d1w1, d1b1, d1w2, d1b2,
              a1w1, a1b1, a1w2, a1b2, a0w1, a0b1, a0w2, a0b2]

    HW, HOWO = _H * _W, _HO * _WO
    out = pl.pallas_call(
        _unet_body,
        out_shape=jax.ShapeDtypeStruct((N, _H, _W, _COUT), F32),
        grid=(N,),
        in_specs=in_specs,
        out_specs=pl.BlockSpec((1, _H, _W, _COUT), lambda n: (n, 0, 0, 0)),
        scratch_shapes=[
            pltpu.VMEM(((_H + 2) * _W, _CIN), BF),
            pltpu.VMEM((HW, 3 * _CIN), BF),
            pltpu.VMEM(((_H + 2) * _W, _C0), BF),
            pltpu.VMEM((HW, 3 * _C0), BF),
            pltpu.VMEM(((_HO + 2) * _WO, _C1), BF),
            pltpu.VMEM((HOWO, 3 * _C1), BF),
            pltpu.VMEM(((_H + 2) * _W, _C1), BF),
            pltpu.VMEM((HW, 3 * _C1), BF),
            pltpu.VMEM((HW, _C0), F32),
            pltpu.VMEM((HW, _C0), F32),
            pltpu.VMEM((HOWO, _C1), F32),
        ],
        compiler_params=pltpu.CompilerParams(
            dimension_semantics=("parallel",),
            vmem_limit_bytes=100 * 1024 * 1024),
    )(*inputs)

    return jnp.transpose(out, (0, 3, 1, 2))
